# baseline (speedup 1.0000x reference)
import functools

import jax
import jax.numpy as jnp
from jax import lax
from jax.experimental import pallas as pl
from jax.experimental.pallas import tpu as pltpu
from jax.experimental.pallas import tpu_sc as plsc

B = 16384
L = 25
H = 64
VOCAB = 1000000

NC = 2
NS = 16
NW = NC * NS
CPW = B // NW
PPW = CPW * L
CCH = 8
PCH = CCH * L
VS0 = 104
VS1 = PCH - VS0
NCH = CPW // CCH
NBUF = 2
W = 2 * H

_mesh = plsc.VectorSubcoreMesh(core_axis_name="c", subcore_axis_name="s")


@functools.partial(
    pl.kernel,
    out_type=jax.ShapeDtypeStruct((B * L,), jnp.float32),
    mesh=_mesh,
    compiler_params=pltpu.CompilerParams(needs_layout_passes=False),
    scratch_types=[
        pltpu.VMEM((CPW + 16,), jnp.int32),
        pltpu.VMEM((CPW,), jnp.int32),
        pltpu.VMEM((PPW + 16,), jnp.int32),
        pltpu.VMEM((PPW,), jnp.int32),
        pltpu.VMEM((NBUF, CCH, W), jnp.float32),
        pltpu.VMEM((NBUF, PCH, W), jnp.float32),
        pltpu.VMEM((PPW,), jnp.float32),
        pltpu.SemaphoreType.DMA((NBUF,)),
        pltpu.SemaphoreType.DMA((NBUF,)),
    ],
)
def _skipgram_sc(cen_hbm, ctx_hbm, u_hbm, v_hbm, out_hbm,
                 cen_v, cenl_v, ctx_v, ctxl_v, ub, vbf, out_v,
                 usem, vsem):
    wid = lax.axis_index("s") * NC + lax.axis_index("c")
    base_c = pl.multiple_of(wid * CPW, CPW)
    base_p = pl.multiple_of(wid * PPW, PPW)

    pltpu.sync_copy(cen_hbm.at[pl.ds(base_c, CPW)], cen_v.at[pl.ds(0, CPW)])
    pltpu.sync_copy(ctx_hbm.at[pl.ds(base_p, PPW)], ctx_v.at[pl.ds(0, PPW)])

    def shift_cen(i, carry):
        cenl_v[pl.ds(i * 16, 16)] = lax.shift_right_logical(
            cen_v[pl.ds(i * 16, 16)], 1)
        return carry

    lax.fori_loop(0, CPW // 16, shift_cen, 0)

    def shift_ctx(i, carry):
        ctxl_v[pl.ds(i * 16, 16)] = lax.shift_right_logical(
            ctx_v[pl.ds(i * 16, 16)], 1)
        return carry

    lax.fori_loop(0, PPW // 16, shift_ctx, 0)

    lanes = lax.iota(jnp.int32, 16)
    mask15 = lanes == 15
    one = jnp.full((16,), 1, jnp.int32)

    def fire(ch, b):
        pltpu.async_copy(
            u_hbm.at[cenl_v.at[pl.ds(ch * CCH, CCH)]], ub.at[b], usem.at[b])
        for off, sz in ((0, 56), (56, 48), (104, 48), (152, 48)):
            pltpu.async_copy(
                v_hbm.at[ctxl_v.at[pl.ds(ch * PCH + off, sz)]],
                vbf.at[b].at[pl.ds(off, sz)], vsem.at[b])

    def drain(b):
        pltpu.make_async_copy(
            u_hbm.at[pl.ds(0, CCH)], ub.at[b], usem.at[b]).wait()
        pltpu.make_async_copy(
            v_hbm.at[pl.ds(0, PCH)], vbf.at[b], vsem.at[b]).wait()

    for b in range(NBUF):
        fire(b, b)

    def compute(ch, b):
        ubb = ub.at[b]
        vb = vbf.at[b]
        cpar = (cen_v[pl.ds(ch * CCH, 16)] & one) * 64
        us = None
        cur = -1
        for g in range(pl.cdiv(PCH, 16)):
            vpar = (ctx_v[pl.ds(ch * PCH + g * 16, 16)] & one) * 64
            for j in range(min(16, PCH - g * 16)):
                r = g * 16 + j
                i = r // L
                if i != cur:
                    cur = i
                    coff = pl.multiple_of(cpar[i], 64)
                    us = [ubb[i, pl.ds(coff + k * 16, 16)]
                          for k in range(H // 16)]
                voff = pl.multiple_of(vpar[j], 64)
                acc = vb[r, pl.ds(voff, 16)] * us[0]
                for k in range(1, H // 16):
                    acc = acc + vb[r, pl.ds(voff + k * 16, 16)] * us[k]
                cum = plsc.cumsum(acc)
                p = ch * PCH + r
                plsc.store_scatter(
                    out_v, [jnp.full((16,), p, jnp.int32)], cum,
                    mask=mask15)

    def group(g, carry):
        for b in range(NBUF):
            ch = g * NBUF + b
            drain(b)
            compute(ch, b)

            @pl.when(ch + NBUF < NCH)
            def _():
                fire(ch + NBUF, b)
        return carry

    lax.fori_loop(0, NCH // NBUF, group, 0)

    for ch in range((NCH // NBUF) * NBUF, NCH):
        b = ch % NBUF
        drain(b)
        compute(ch, b)

    pltpu.sync_copy(out_v, out_hbm.at[pl.ds(base_p, PPW)])


def kernel(center_ids, context_neg_ids, U, V):
    cen = center_ids.reshape(-1).astype(jnp.int32)
    ctx = context_neg_ids.reshape(-1).astype(jnp.int32)
    u2 = U.reshape(VOCAB // 2, 2 * H)
    v2 = V.reshape(VOCAB // 2, 2 * H)
    out = _skipgram_sc(cen, ctx, u2, v2)
    return out.reshape(B, L)


# speedup vs baseline: 1.0952x; 1.0952x over previous
import functools

import jax
import jax.numpy as jnp
from jax import lax
from jax.experimental import pallas as pl
from jax.experimental.pallas import tpu as pltpu
from jax.experimental.pallas import tpu_sc as plsc

B = 16384
L = 25
H = 64
VOCAB = 1000000

NC = 2
NS = 16
NW = NC * NS
CPW = B // NW
PPW = CPW * L
CCH = 8
PCH = CCH * L
VS0 = 104
VS1 = PCH - VS0
NCH = CPW // CCH
NBUF = 3
W = 2 * H

_mesh = plsc.VectorSubcoreMesh(core_axis_name="c", subcore_axis_name="s")


@functools.partial(
    pl.kernel,
    out_type=jax.ShapeDtypeStruct((B * L,), jnp.float32),
    mesh=_mesh,
    compiler_params=pltpu.CompilerParams(needs_layout_passes=False),
    scratch_types=[
        pltpu.VMEM((CPW + 16,), jnp.int32),
        pltpu.VMEM((CPW,), jnp.int32),
        pltpu.VMEM((PPW + 16,), jnp.int32),
        pltpu.VMEM((PPW,), jnp.int32),
        pltpu.VMEM((NBUF, CCH, W), jnp.float32),
        pltpu.VMEM((NBUF, PCH, W), jnp.float32),
        pltpu.VMEM((CCH * H,), jnp.float32),
        pltpu.VMEM((PPW,), jnp.float32),
        pltpu.SemaphoreType.DMA((NBUF,)),
        pltpu.SemaphoreType.DMA((NBUF,)),
    ],
)
def _skipgram_sc(cen_hbm, ctx_hbm, u_hbm, v_hbm, out_hbm,
                 cen_v, cenl_v, ctx_v, ctxl_v, ub, vbf, ucan, out_v,
                 usem, vsem):
    wid = lax.axis_index("s") * NC + lax.axis_index("c")
    base_c = pl.multiple_of(wid * CPW, CPW)
    base_p = pl.multiple_of(wid * PPW, PPW)

    pltpu.sync_copy(cen_hbm.at[pl.ds(base_c, CPW)], cen_v.at[pl.ds(0, CPW)])
    pltpu.sync_copy(ctx_hbm.at[pl.ds(base_p, PPW)], ctx_v.at[pl.ds(0, PPW)])

    def shift_cen(i, carry):
        cenl_v[pl.ds(i * 16, 16)] = lax.shift_right_logical(
            cen_v[pl.ds(i * 16, 16)], 1)
        return carry

    lax.fori_loop(0, CPW // 16, shift_cen, 0)

    def shift_ctx(i, carry):
        ctxl_v[pl.ds(i * 16, 16)] = lax.shift_right_logical(
            ctx_v[pl.ds(i * 16, 16)], 1)
        return carry

    lax.fori_loop(0, PPW // 16, shift_ctx, 0)

    lanes = lax.iota(jnp.int32, 16)
    mask15 = lanes == 15
    one = jnp.full((16,), 1, jnp.int32)

    def fire(ch, b):
        pltpu.async_copy(
            u_hbm.at[cenl_v.at[pl.ds(ch * CCH, CCH)]], ub.at[b], usem.at[b])
        for off, sz in ((0, 56), (56, 48), (104, 48), (152, 48)):
            pltpu.async_copy(
                v_hbm.at[ctxl_v.at[pl.ds(ch * PCH + off, sz)]],
                vbf.at[b].at[pl.ds(off, sz)], vsem.at[b])

    def drain(b):
        pltpu.make_async_copy(
            u_hbm.at[pl.ds(0, CCH)], ub.at[b], usem.at[b]).wait()
        pltpu.make_async_copy(
            v_hbm.at[pl.ds(0, PCH)], vbf.at[b], vsem.at[b]).wait()

    for b in range(NBUF):
        fire(b, b)

    def compute(ch, b):
        ubb = ub.at[b]
        vb = vbf.at[b]
        cpar = (cen_v[pl.ds(ch * CCH, 16)] & one) * 64
        for i in range(CCH):
            coff = pl.multiple_of(cpar[i], 64)
            for k in range(H // 16):
                ucan[pl.ds(i * H + k * 16, 16)] = (
                    ubb[i, pl.ds(coff + k * 16, 16)])

        def pair(r, vpar, j, base):
            ioff = pl.multiple_of(
                lax.shift_right_logical(r * 41, 10) * H, H)
            us = [ucan[pl.ds(ioff + k * 16, 16)] for k in range(H // 16)]
            voff = pl.multiple_of(vpar[j], 64)
            acc = vb[r, pl.ds(voff, 16)] * us[0]
            for k in range(1, H // 16):
                acc = acc + vb[r, pl.ds(voff + k * 16, 16)] * us[k]
            cum = plsc.cumsum(acc)
            plsc.store_scatter(
                out_v, [jnp.full((16,), base + j, jnp.int32)], cum,
                mask=mask15)

        def gbody(g, carry):
            base = ch * PCH + g * 16
            vpar = (ctx_v[pl.ds(base, 16)] & one) * 64
            for j in range(16):
                pair(g * 16 + j, vpar, j, base)
            return carry

        lax.fori_loop(0, PCH // 16, gbody, 0)

        base = ch * PCH + (PCH // 16) * 16
        vpar = (ctx_v[pl.ds(base, 16)] & one) * 64
        for j in range(PCH - (PCH // 16) * 16):
            pair((PCH // 16) * 16 + j, vpar, j, base)

    def group(g, carry):
        for b in range(NBUF):
            ch = g * NBUF + b
            drain(b)
            compute(ch, b)

            @pl.when(ch + NBUF < NCH)
            def _():
                fire(ch + NBUF, b)
        return carry

    lax.fori_loop(0, NCH // NBUF, group, 0)

    for ch in range((NCH // NBUF) * NBUF, NCH):
        b = ch % NBUF
        drain(b)
        compute(ch, b)

    pltpu.sync_copy(out_v, out_hbm.at[pl.ds(base_p, PPW)])


def kernel(center_ids, context_neg_ids, U, V):
    cen = center_ids.reshape(-1).astype(jnp.int32)
    ctx = context_neg_ids.reshape(-1).astype(jnp.int32)
    u2 = U.reshape(VOCAB // 2, 2 * H)
    v2 = V.reshape(VOCAB // 2, 2 * H)
    out = _skipgram_sc(cen, ctx, u2, v2)
    return out.reshape(B, L)


# speedup vs baseline: 1.1041x; 1.0081x over previous
import functools

import jax
import jax.numpy as jnp
from jax import lax
from jax.experimental import pallas as pl
from jax.experimental.pallas import tpu as pltpu
from jax.experimental.pallas import tpu_sc as plsc

B = 16384
L = 25
H = 64
VOCAB = 1000000

NC = 2
NS = 16
NW = NC * NS
CPW = B // NW
PPW = CPW * L
CCH = 8
PCH = CCH * L
VS0 = 104
VS1 = PCH - VS0
NCH = CPW // CCH
NBUF = 3
W = 2 * H

_mesh = plsc.VectorSubcoreMesh(core_axis_name="c", subcore_axis_name="s")


@functools.partial(
    pl.kernel,
    out_type=jax.ShapeDtypeStruct((B * L,), jnp.float32),
    mesh=_mesh,
    compiler_params=pltpu.CompilerParams(needs_layout_passes=False),
    scratch_types=[
        pltpu.VMEM((CPW + 16,), jnp.int32),
        pltpu.VMEM((CPW,), jnp.int32),
        pltpu.VMEM((PPW + 16,), jnp.int32),
        pltpu.VMEM((PPW,), jnp.int32),
        pltpu.VMEM((NBUF, CCH, W), jnp.float32),
        pltpu.VMEM((NBUF, PCH, W), jnp.float32),
        pltpu.VMEM((CCH * H + H,), jnp.float32),
        pltpu.VMEM((PPW,), jnp.float32),
        pltpu.SemaphoreType.DMA((NBUF,)),
        pltpu.SemaphoreType.DMA((NBUF,)),
    ],
)
def _skipgram_sc(cen_hbm, ctx_hbm, u_hbm, v_hbm, out_hbm,
                 cen_v, cenl_v, ctx_v, ctxl_v, ub, vbf, ucan, out_v,
                 usem, vsem):
    wid = lax.axis_index("s") * NC + lax.axis_index("c")
    base_c = pl.multiple_of(wid * CPW, CPW)
    base_p = pl.multiple_of(wid * PPW, PPW)

    pltpu.sync_copy(cen_hbm.at[pl.ds(base_c, CPW)], cen_v.at[pl.ds(0, CPW)])
    pltpu.sync_copy(ctx_hbm.at[pl.ds(base_p, PPW)], ctx_v.at[pl.ds(0, PPW)])

    def shift_cen(i, carry):
        cenl_v[pl.ds(i * 16, 16)] = lax.shift_right_logical(
            cen_v[pl.ds(i * 16, 16)], 1)
        return carry

    lax.fori_loop(0, CPW // 16, shift_cen, 0)

    def shift_ctx(i, carry):
        ctxl_v[pl.ds(i * 16, 16)] = lax.shift_right_logical(
            ctx_v[pl.ds(i * 16, 16)], 1)
        return carry

    lax.fori_loop(0, PPW // 16, shift_ctx, 0)

    lanes = lax.iota(jnp.int32, 16)
    mask15 = lanes == 15
    one = jnp.full((16,), 1, jnp.int32)

    def fire(ch, b):
        pltpu.async_copy(
            u_hbm.at[cenl_v.at[pl.ds(ch * CCH, CCH)]], ub.at[b], usem.at[b])
        for off, sz in ((0, 56), (56, 48), (104, 48), (152, 48)):
            pltpu.async_copy(
                v_hbm.at[ctxl_v.at[pl.ds(ch * PCH + off, sz)]],
                vbf.at[b].at[pl.ds(off, sz)], vsem.at[b])

    def drain(b):
        pltpu.make_async_copy(
            u_hbm.at[pl.ds(0, CCH)], ub.at[b], usem.at[b]).wait()
        pltpu.make_async_copy(
            v_hbm.at[pl.ds(0, PCH)], vbf.at[b], vsem.at[b]).wait()

    for b in range(NBUF):
        fire(b, b)

    def compute(ch, b):
        ubb = ub.at[b]
        vb = vbf.at[b]
        cpar = (cen_v[pl.ds(ch * CCH, 16)] & one) * 64
        for i in range(CCH):
            coff = pl.multiple_of(cpar[i], 64)
            for k in range(H // 16):
                ucan[pl.ds(i * H + k * 16, 16)] = (
                    ubb[i, pl.ds(coff + k * 16, 16)])

        def pair(r, vpar, j, base):
            ioff = pl.multiple_of(
                lax.shift_right_logical(r * 41, 10) * H, H)
            us = [ucan[pl.ds(ioff + k * 16, 16)] for k in range(H // 16)]
            voff = pl.multiple_of(vpar[j], 64)
            acc = vb[r, pl.ds(voff, 16)] * us[0]
            for k in range(1, H // 16):
                acc = acc + vb[r, pl.ds(voff + k * 16, 16)] * us[k]
            cum = plsc.cumsum(acc)
            plsc.store_scatter(
                out_v, [jnp.full((16,), base + j, jnp.int32)], cum,
                mask=mask15)

        def gbody(g, carry):
            base = ch * PCH + g * 16
            vpar = (ctx_v[pl.ds(base, 16)] & one) * 64
            r0 = g * 16
            i0 = lax.shift_right_logical(r0 * 41, 10)
            o0 = pl.multiple_of(i0 * H, H)
            o1 = pl.multiple_of((i0 + 1) * H, H)
            us0 = [ucan[pl.ds(o0 + k * 16, 16)] for k in range(H // 16)]
            us1 = [ucan[pl.ds(o1 + k * 16, 16)] for k in range(H // 16)]
            t = (i0 + 1) * L - r0
            for j in range(16):
                r = r0 + j
                sel = jnp.full((16,), j >= t, jnp.bool_)
                us = [jnp.where(sel, us1[k], us0[k])
                      for k in range(H // 16)]
                voff = pl.multiple_of(vpar[j], 64)
                acc = vb[r, pl.ds(voff, 16)] * us[0]
                for k in range(1, H // 16):
                    acc = acc + vb[r, pl.ds(voff + k * 16, 16)] * us[k]
                cum = plsc.cumsum(acc)
                plsc.store_scatter(
                    out_v, [jnp.full((16,), base + j, jnp.int32)], cum,
                    mask=mask15)
            return carry

        lax.fori_loop(0, PCH // 16, gbody, 0)

        base = ch * PCH + (PCH // 16) * 16
        vpar = (ctx_v[pl.ds(base, 16)] & one) * 64
        for j in range(PCH - (PCH // 16) * 16):
            pair((PCH // 16) * 16 + j, vpar, j, base)

    def group(g, carry):
        for b in range(NBUF):
            ch = g * NBUF + b
            drain(b)
            compute(ch, b)

            @pl.when(ch + NBUF < NCH)
            def _():
                fire(ch + NBUF, b)
        return carry

    lax.fori_loop(0, NCH // NBUF, group, 0)

    for ch in range((NCH // NBUF) * NBUF, NCH):
        b = ch % NBUF
        drain(b)
        compute(ch, b)

    pltpu.sync_copy(out_v, out_hbm.at[pl.ds(base_p, PPW)])


def kernel(center_ids, context_neg_ids, U, V):
    cen = center_ids.reshape(-1).astype(jnp.int32)
    ctx = context_neg_ids.reshape(-1).astype(jnp.int32)
    u2 = U.reshape(VOCAB // 2, 2 * H)
    v2 = V.reshape(VOCAB // 2, 2 * H)
    out = _skipgram_sc(cen, ctx, u2, v2)
    return out.reshape(B, L)


# speedup vs baseline: 1.1053x; 1.0010x over previous
import functools

import jax
import jax.numpy as jnp
from jax import lax
from jax.experimental import pallas as pl
from jax.experimental.pallas import tpu as pltpu
from jax.experimental.pallas import tpu_sc as plsc

B = 16384
L = 25
H = 64
VOCAB = 1000000

NC = 2
NS = 16
NW = NC * NS
CPW = B // NW
PPW = CPW * L
CCH = 8
PCH = CCH * L
VS0 = 104
VS1 = PCH - VS0
NCH = CPW // CCH
NBUF = 3
W = 2 * H

_mesh = plsc.VectorSubcoreMesh(core_axis_name="c", subcore_axis_name="s")


@functools.partial(
    pl.kernel,
    out_type=jax.ShapeDtypeStruct((B * L,), jnp.float32),
    mesh=_mesh,
    compiler_params=pltpu.CompilerParams(needs_layout_passes=False),
    scratch_types=[
        pltpu.VMEM((CPW + 16,), jnp.int32),
        pltpu.VMEM((CPW,), jnp.int32),
        pltpu.VMEM((PPW + 16,), jnp.int32),
        pltpu.VMEM((PPW,), jnp.int32),
        pltpu.VMEM((NBUF, CCH, W), jnp.float32),
        pltpu.VMEM((NBUF, PCH, W), jnp.float32),
        pltpu.VMEM((CCH * H + H,), jnp.float32),
        pltpu.VMEM((PPW,), jnp.float32),
        pltpu.SemaphoreType.DMA((NBUF,)),
        pltpu.SemaphoreType.DMA((NBUF,)),
    ],
)
def _skipgram_sc(cen_hbm, ctx_hbm, u_hbm, v_hbm, out_hbm,
                 cen_v, cenl_v, ctx_v, ctxl_v, ub, vbf, ucan, out_v,
                 usem, vsem):
    wid = lax.axis_index("s") * NC + lax.axis_index("c")
    base_c = pl.multiple_of(wid * CPW, CPW)
    base_p = pl.multiple_of(wid * PPW, PPW)

    pltpu.sync_copy(cen_hbm.at[pl.ds(base_c, CPW)], cen_v.at[pl.ds(0, CPW)])
    pltpu.sync_copy(ctx_hbm.at[pl.ds(base_p, PPW)], ctx_v.at[pl.ds(0, PPW)])

    def shift_cen(i, carry):
        cenl_v[pl.ds(i * 16, 16)] = lax.shift_right_logical(
            cen_v[pl.ds(i * 16, 16)], 1)
        return carry

    lax.fori_loop(0, CPW // 16, shift_cen, 0)

    def shift_ctx(i, carry):
        ctxl_v[pl.ds(i * 16, 16)] = lax.shift_right_logical(
            ctx_v[pl.ds(i * 16, 16)], 1)
        return carry

    lax.fori_loop(0, PPW // 16, shift_ctx, 0)

    lanes = lax.iota(jnp.int32, 16)
    mask15 = lanes == 15
    one = jnp.full((16,), 1, jnp.int32)

    def fire(ch, b):
        pltpu.async_copy(
            u_hbm.at[cenl_v.at[pl.ds(ch * CCH, CCH)]], ub.at[b], usem.at[b])
        for off, sz in ((0, 24), (24, 24), (48, 24), (72, 24),
                        (96, 24), (120, 24), (144, 24), (168, 32)):
            pltpu.async_copy(
                v_hbm.at[ctxl_v.at[pl.ds(ch * PCH + off, sz)]],
                vbf.at[b].at[pl.ds(off, sz)], vsem.at[b])

    def drain(b):
        pltpu.make_async_copy(
            u_hbm.at[pl.ds(0, CCH)], ub.at[b], usem.at[b]).wait()
        pltpu.make_async_copy(
            v_hbm.at[pl.ds(0, PCH)], vbf.at[b], vsem.at[b]).wait()

    for b in range(NBUF):
        fire(b, b)

    def compute(ch, b):
        ubb = ub.at[b]
        vb = vbf.at[b]
        cpar = (cen_v[pl.ds(ch * CCH, 16)] & one) * 64
        for i in range(CCH):
            coff = pl.multiple_of(cpar[i], 64)
            for k in range(H // 16):
                ucan[pl.ds(i * H + k * 16, 16)] = (
                    ubb[i, pl.ds(coff + k * 16, 16)])

        def pair(r, vpar, j, base):
            ioff = pl.multiple_of(
                lax.shift_right_logical(r * 41, 10) * H, H)
            us = [ucan[pl.ds(ioff + k * 16, 16)] for k in range(H // 16)]
            voff = pl.multiple_of(vpar[j], 64)
            acc = vb[r, pl.ds(voff, 16)] * us[0]
            for k in range(1, H // 16):
                acc = acc + vb[r, pl.ds(voff + k * 16, 16)] * us[k]
            cum = plsc.cumsum(acc)
            plsc.store_scatter(
                out_v, [jnp.full((16,), base + j, jnp.int32)], cum,
                mask=mask15)

        def gbody(g, carry):
            base = ch * PCH + g * 16
            vpar = (ctx_v[pl.ds(base, 16)] & one) * 64
            r0 = g * 16
            i0 = lax.shift_right_logical(r0 * 41, 10)
            o0 = pl.multiple_of(i0 * H, H)
            o1 = pl.multiple_of((i0 + 1) * H, H)
            us0 = [ucan[pl.ds(o0 + k * 16, 16)] for k in range(H // 16)]
            us1 = [ucan[pl.ds(o1 + k * 16, 16)] for k in range(H // 16)]
            t = (i0 + 1) * L - r0
            for j in range(16):
                r = r0 + j
                sel = jnp.full((16,), j >= t, jnp.bool_)
                us = [jnp.where(sel, us1[k], us0[k])
                      for k in range(H // 16)]
                voff = pl.multiple_of(vpar[j], 64)
                acc = vb[r, pl.ds(voff, 16)] * us[0]
                for k in range(1, H // 16):
                    acc = acc + vb[r, pl.ds(voff + k * 16, 16)] * us[k]
                cum = plsc.cumsum(acc)
                plsc.store_scatter(
                    out_v, [jnp.full((16,), base + j, jnp.int32)], cum,
                    mask=mask15)
            return carry

        lax.fori_loop(0, PCH // 16, gbody, 0)

        base = ch * PCH + (PCH // 16) * 16
        vpar = (ctx_v[pl.ds(base, 16)] & one) * 64
        for j in range(PCH - (PCH // 16) * 16):
            pair((PCH // 16) * 16 + j, vpar, j, base)

    def group(g, carry):
        for b in range(NBUF):
            ch = g * NBUF + b
            drain(b)
            compute(ch, b)

            @pl.when(ch + NBUF < NCH)
            def _():
                fire(ch + NBUF, b)
        return carry

    lax.fori_loop(0, NCH // NBUF, group, 0)

    for ch in range((NCH // NBUF) * NBUF, NCH):
        b = ch % NBUF
        drain(b)
        compute(ch, b)

    pltpu.sync_copy(out_v, out_hbm.at[pl.ds(base_p, PPW)])


def kernel(center_ids, context_neg_ids, U, V):
    cen = center_ids.reshape(-1).astype(jnp.int32)
    ctx = context_neg_ids.reshape(-1).astype(jnp.int32)
    u2 = U.reshape(VOCAB // 2, 2 * H)
    v2 = V.reshape(VOCAB // 2, 2 * H)
    out = _skipgram_sc(cen, ctx, u2, v2)
    return out.reshape(B, L)
